# Initial kernel scaffold; baseline (speedup 1.0000x reference)
import jax
import jax.numpy as jnp
from jax.experimental import pallas as pl


def kernel(x, edge_index, batch, W_rel0, b_rel0, W_root0, W_rel1, b_rel1, W_root1, W_rel2, b_rel2, W_root2, lin1_W, lin1_b, lin2_W, lin2_b, lin3_W, lin3_b):
    raise NotImplementedError("write your pallas kernel here")


# speedup vs baseline: 4.7572x; 4.7572x over previous
import functools

import jax
import jax.numpy as jnp
from jax import lax
from jax.experimental import pallas as pl
from jax.experimental.pallas import tpu as pltpu
from jax.experimental.pallas import tpu_sc as plsc

N = 10000
E = 320000
D = 128
G = 64
OUT = 10

NC = 2
NS = 16
NW = NC * NS

CHUNK = 128
EPT = E // NW
FULL = EPT // CHUNK
REM = EPT - FULL * CHUNK
NP = 10240
RPT = NP // NS

def _seg_sum_body(h_hbm, src_hbm, dst_hbm, out_hbm,
                  src_v, dst_v, srcr_v, dstr_v, rows_v, acc_sh, sem):
    cid = lax.axis_index("c")
    sid = lax.axis_index("s")
    wid = cid * NS + sid

    zz = jnp.zeros((16,), jnp.float32)

    def _zrow(r, carry):
        for j in range(D // 16):
            rows_v[r, pl.ds(j * 16, 16)] = zz
        return carry

    lax.fori_loop(0, CHUNK, _zrow, 0)
    for t in range(RPT // CHUNK):
        pltpu.sync_copy(rows_v,
                        acc_sh.at[pl.ds(sid * RPT + t * CHUNK, CHUNK)])
    plsc.subcore_barrier()

    gbase = wid * EPT

    def _chunk(ci, carry):
        base = pl.multiple_of(gbase + ci * CHUNK, 8)
        pltpu.sync_copy(src_hbm.at[pl.ds(base, CHUNK)], src_v)
        pltpu.sync_copy(dst_hbm.at[pl.ds(base, CHUNK)], dst_v)
        pltpu.async_copy(h_hbm.at[src_v], rows_v, sem).wait()
        pltpu.sync_copy(rows_v, acc_sh.at[dst_v], add=True)
        return carry

    lax.fori_loop(0, FULL, _chunk, 0)

    base = gbase + FULL * CHUNK
    pltpu.sync_copy(src_hbm.at[pl.ds(base, REM)], srcr_v)
    pltpu.sync_copy(dst_hbm.at[pl.ds(base, REM)], dstr_v)
    pltpu.async_copy(h_hbm.at[srcr_v], rows_v.at[pl.ds(0, REM)], sem).wait()
    pltpu.sync_copy(rows_v.at[pl.ds(0, REM)], acc_sh.at[dstr_v], add=True)

    plsc.subcore_barrier()
    pltpu.sync_copy(acc_sh.at[pl.ds(sid * RPT, RPT)],
                    out_hbm.at[cid, pl.ds(sid * RPT, RPT)])


@functools.cache
def _get_seg_sum():
    mesh = plsc.VectorSubcoreMesh(
        core_axis_name="c", subcore_axis_name="s",
        num_cores=NC, num_subcores=NS)
    return pl.kernel(
        _seg_sum_body,
        out_type=jax.ShapeDtypeStruct((NC, NP, D), jnp.float32),
        mesh=mesh,
        scratch_types=[
            pltpu.VMEM((CHUNK,), jnp.int32),
            pltpu.VMEM((CHUNK,), jnp.int32),
            pltpu.VMEM((REM,), jnp.int32),
            pltpu.VMEM((REM,), jnp.int32),
            pltpu.VMEM((CHUNK, D), jnp.float32),
            pltpu.VMEM_SHARED((NP, D), jnp.float32),
            pltpu.SemaphoreType.DMA,
        ],
    )


def _seg_sum(h, src, dst):
    return _get_seg_sum()(h, src, dst)


BN = 1000
NB = N // BN


def _layer_body(acc_ref, h_ref, batch_ref, wr_ref, br_ref, wo_ref,
                hn_ref, ps_ref, pm_ref, cnt_ref):
    i = pl.program_id(0)
    agg = acc_ref[0] + acc_ref[1]
    hn = jnp.dot(agg, wr_ref[...], preferred_element_type=jnp.float32)
    hn = hn + jnp.dot(h_ref[...], wo_ref[...], preferred_element_type=jnp.float32)
    hn = jnp.maximum(hn + br_ref[...], 0.0)
    hn_ref[...] = hn

    bb = batch_ref[...][:, 0]
    onehot = (bb[None, :] == lax.broadcasted_iota(jnp.int32, (G, BN), 0)
              ).astype(jnp.float32)
    contrib = jnp.dot(onehot, hn, preferred_element_type=jnp.float32)
    ccnt = jnp.dot(onehot, jnp.ones((BN, D), jnp.float32),
                   preferred_element_type=jnp.float32)

    @pl.when(i == 0)
    def _():
        ps_ref[...] = jnp.zeros((G, D), jnp.float32)
        pm_ref[...] = jnp.zeros((G, D), jnp.float32)
        cnt_ref[...] = jnp.zeros((G, D), jnp.float32)

    ps_ref[...] += contrib
    cnt_ref[...] += ccnt

    lo = bb[0]
    hi = bb[BN - 1]
    gcol = lax.broadcasted_iota(jnp.int32, (G, 1), 0)

    def _g(g, cur):
        cand = jnp.max(jnp.where((bb == g)[:, None], hn, 0.0), axis=0)
        return jnp.maximum(cur, jnp.where(gcol == g, cand[None, :], 0.0))

    pm_ref[...] = lax.fori_loop(lo, hi + 1, _g, pm_ref[...])


def _layer_call(acc, h, batch2d, wr, br, wo):
    return pl.pallas_call(
        _layer_body,
        grid=(NB,),
        in_specs=[
            pl.BlockSpec((NC, BN, D), lambda i: (0, i, 0)),
            pl.BlockSpec((BN, D), lambda i: (i, 0)),
            pl.BlockSpec((BN, 1), lambda i: (i, 0)),
            pl.BlockSpec((D, D), lambda i: (0, 0)),
            pl.BlockSpec((1, D), lambda i: (0, 0)),
            pl.BlockSpec((D, D), lambda i: (0, 0)),
        ],
        out_specs=[
            pl.BlockSpec((BN, D), lambda i: (i, 0)),
            pl.BlockSpec((G, D), lambda i: (0, 0)),
            pl.BlockSpec((G, D), lambda i: (0, 0)),
            pl.BlockSpec((G, D), lambda i: (0, 0)),
        ],
        out_shape=[
            jax.ShapeDtypeStruct((N, D), jnp.float32),
            jax.ShapeDtypeStruct((G, D), jnp.float32),
            jax.ShapeDtypeStruct((G, D), jnp.float32),
            jax.ShapeDtypeStruct((G, D), jnp.float32),
        ],
    )(acc, h, batch2d, wr, br, wo)


def _head_body(s1, s2, s3, m1, m2, m3, cnt,
               w1, b1, w2, b2, w3, b3, out_ref):
    S = s1[...] + s2[...] + s3[...]
    M = m1[...] + m2[...] + m3[...]
    mean = S / jnp.maximum(cnt[...], 1.0)
    z = jnp.concatenate([mean, M], axis=1)
    z = jnp.maximum(
        jnp.dot(z, w1[...], preferred_element_type=jnp.float32) + b1[...], 0.0)
    z = jnp.maximum(
        jnp.dot(z, w2[...], preferred_element_type=jnp.float32) + b2[...], 0.0)
    out_ref[...] = (
        jnp.dot(z, w3[...], preferred_element_type=jnp.float32) + b3[...])


def _head_call(s1, s2, s3, m1, m2, m3, cnt, w1, b1, w2, b2, w3, b3):
    return pl.pallas_call(
        _head_body,
        out_shape=jax.ShapeDtypeStruct((G, OUT), jnp.float32),
    )(s1, s2, s3, m1, m2, m3, cnt, w1, b1, w2, b2, w3, b3)


def kernel(x, edge_index, batch,
           W_rel0, b_rel0, W_root0,
           W_rel1, b_rel1, W_root1,
           W_rel2, b_rel2, W_root2,
           lin1_W, lin1_b, lin2_W, lin2_b, lin3_W, lin3_b):
    src = edge_index[0]
    dst = edge_index[1]
    batch2d = batch.reshape(N, 1)

    acc = _seg_sum(x, src, dst)
    h1, s1, m1, cnt = _layer_call(acc, x, batch2d,
                                  W_rel0, b_rel0.reshape(1, D), W_root0)
    acc = _seg_sum(h1, src, dst)
    h2, s2, m2, _ = _layer_call(acc, h1, batch2d,
                                W_rel1, b_rel1.reshape(1, D), W_root1)
    acc = _seg_sum(h2, src, dst)
    h3, s3, m3, _ = _layer_call(acc, h2, batch2d,
                                W_rel2, b_rel2.reshape(1, D), W_root2)

    return _head_call(s1, s2, s3, m1, m2, m3, cnt,
                      lin1_W, lin1_b.reshape(1, G), lin2_W,
                      lin2_b.reshape(1, 32), lin3_W, lin3_b.reshape(1, OUT))


# speedup vs baseline: 7.2465x; 1.5233x over previous
import functools

import jax
import jax.numpy as jnp
from jax import lax
from jax.experimental import pallas as pl
from jax.experimental.pallas import tpu as pltpu
from jax.experimental.pallas import tpu_sc as plsc

N = 10000
E = 320000
D = 128
G = 64
OUT = 10

NC = 2
NS = 16
NW = NC * NS

CPT = 80
EPT = E // NW
NCH = EPT // CPT
GRP = 25
NGRP = NCH // GRP
NP = 10240
RPT = NP // NS

def _seg_sum_body(h_hbm, src_hbm, dst_hbm, out_hbm,
                  src_v, dst_v, rows0, rows1, acc_sh, sem0, sem1):
    cid = lax.axis_index("c")
    sid = lax.axis_index("s")
    wid = cid * NS + sid

    zz = jnp.zeros((16,), jnp.float32)

    def _zrow(r, carry):
        for j in range(D // 16):
            rows0[r, pl.ds(j * 16, 16)] = zz
        return carry

    lax.fori_loop(0, CPT, _zrow, 0)
    for t in range(RPT // CPT):
        pltpu.sync_copy(rows0,
                        acc_sh.at[pl.ds(sid * RPT + t * CPT, CPT)])
    plsc.subcore_barrier()

    for g in range(NGRP):
        pltpu.sync_copy(src_hbm.at[wid, g], src_v)
        pltpu.sync_copy(dst_hbm.at[wid, g], dst_v)

        pltpu.async_copy(h_hbm.at[src_v.at[0]], rows0, sem0)

        def _pair(p, carry):
            c0 = 2 * p
            c1 = c0 + 1
            pltpu.async_copy(h_hbm.at[src_v.at[c1]], rows1, sem1)
            pltpu.make_async_copy(h_hbm.at[src_v.at[c0]], rows0, sem0).wait()
            pltpu.sync_copy(rows0, acc_sh.at[dst_v.at[c0]], add=True)
            pltpu.async_copy(h_hbm.at[src_v.at[c1 + 1]], rows0, sem0)
            pltpu.make_async_copy(h_hbm.at[src_v.at[c1]], rows1, sem1).wait()
            pltpu.sync_copy(rows1, acc_sh.at[dst_v.at[c1]], add=True)
            return carry

        lax.fori_loop(0, (GRP - 1) // 2, _pair, 0)

        pltpu.make_async_copy(h_hbm.at[src_v.at[GRP - 1]], rows0, sem0).wait()
        pltpu.sync_copy(rows0, acc_sh.at[dst_v.at[GRP - 1]], add=True)

    plsc.subcore_barrier()
    pltpu.sync_copy(acc_sh.at[pl.ds(sid * RPT, RPT)],
                    out_hbm.at[cid, pl.ds(sid * RPT, RPT)])


@functools.cache
def _get_seg_sum():
    mesh = plsc.VectorSubcoreMesh(
        core_axis_name="c", subcore_axis_name="s",
        num_cores=NC, num_subcores=NS)
    return pl.kernel(
        _seg_sum_body,
        out_type=jax.ShapeDtypeStruct((NC, NP, D), jnp.float32),
        mesh=mesh,
        scratch_types=[
            pltpu.VMEM((GRP, CPT), jnp.int32),
            pltpu.VMEM((GRP, CPT), jnp.int32),
            pltpu.VMEM((CPT, D), jnp.float32),
            pltpu.VMEM((CPT, D), jnp.float32),
            pltpu.VMEM_SHARED((NP, D), jnp.float32),
            pltpu.SemaphoreType.DMA,
            pltpu.SemaphoreType.DMA,
        ],
    )


def _seg_sum(h, src3, dst3):
    return _get_seg_sum()(h, src3, dst3)


BN = 1000
NB = N // BN


def _layer_body(acc_ref, h_ref, batch_ref, wr_ref, br_ref, wo_ref,
                hn_ref, ps_ref, pm_ref, cnt_ref):
    i = pl.program_id(0)
    agg = acc_ref[0] + acc_ref[1]
    hn = jnp.dot(agg, wr_ref[...], preferred_element_type=jnp.float32)
    hn = hn + jnp.dot(h_ref[...], wo_ref[...], preferred_element_type=jnp.float32)
    hn = jnp.maximum(hn + br_ref[...], 0.0)
    hn_ref[...] = hn

    bb = batch_ref[...][:, 0]
    onehot = (bb[None, :] == lax.broadcasted_iota(jnp.int32, (G, BN), 0)
              ).astype(jnp.float32)
    contrib = jnp.dot(onehot, hn, preferred_element_type=jnp.float32)
    ccnt = jnp.dot(onehot, jnp.ones((BN, D), jnp.float32),
                   preferred_element_type=jnp.float32)

    @pl.when(i == 0)
    def _():
        ps_ref[...] = jnp.zeros((G, D), jnp.float32)
        pm_ref[...] = jnp.zeros((G, D), jnp.float32)
        cnt_ref[...] = jnp.zeros((G, D), jnp.float32)

    ps_ref[...] += contrib
    cnt_ref[...] += ccnt

    lo = bb[0]
    hi = bb[BN - 1]
    gcol = lax.broadcasted_iota(jnp.int32, (G, 1), 0)

    def _g(g, cur):
        cand = jnp.max(jnp.where((bb == g)[:, None], hn, 0.0), axis=0)
        return jnp.maximum(cur, jnp.where(gcol == g, cand[None, :], 0.0))

    pm_ref[...] = lax.fori_loop(lo, hi + 1, _g, pm_ref[...])


def _layer_call(acc, h, batch2d, wr, br, wo):
    return pl.pallas_call(
        _layer_body,
        grid=(NB,),
        in_specs=[
            pl.BlockSpec((NC, BN, D), lambda i: (0, i, 0)),
            pl.BlockSpec((BN, D), lambda i: (i, 0)),
            pl.BlockSpec((BN, 1), lambda i: (i, 0)),
            pl.BlockSpec((D, D), lambda i: (0, 0)),
            pl.BlockSpec((1, D), lambda i: (0, 0)),
            pl.BlockSpec((D, D), lambda i: (0, 0)),
        ],
        out_specs=[
            pl.BlockSpec((BN, D), lambda i: (i, 0)),
            pl.BlockSpec((G, D), lambda i: (0, 0)),
            pl.BlockSpec((G, D), lambda i: (0, 0)),
            pl.BlockSpec((G, D), lambda i: (0, 0)),
        ],
        out_shape=[
            jax.ShapeDtypeStruct((N, D), jnp.float32),
            jax.ShapeDtypeStruct((G, D), jnp.float32),
            jax.ShapeDtypeStruct((G, D), jnp.float32),
            jax.ShapeDtypeStruct((G, D), jnp.float32),
        ],
    )(acc, h, batch2d, wr, br, wo)


def _head_body(s1, s2, s3, m1, m2, m3, cnt,
               w1, b1, w2, b2, w3, b3, out_ref):
    S = s1[...] + s2[...] + s3[...]
    M = m1[...] + m2[...] + m3[...]
    mean = S / jnp.maximum(cnt[...], 1.0)
    z = jnp.concatenate([mean, M], axis=1)
    z = jnp.maximum(
        jnp.dot(z, w1[...], preferred_element_type=jnp.float32) + b1[...], 0.0)
    z = jnp.maximum(
        jnp.dot(z, w2[...], preferred_element_type=jnp.float32) + b2[...], 0.0)
    out_ref[...] = (
        jnp.dot(z, w3[...], preferred_element_type=jnp.float32) + b3[...])


def _head_call(s1, s2, s3, m1, m2, m3, cnt, w1, b1, w2, b2, w3, b3):
    return pl.pallas_call(
        _head_body,
        out_shape=jax.ShapeDtypeStruct((G, OUT), jnp.float32),
    )(s1, s2, s3, m1, m2, m3, cnt, w1, b1, w2, b2, w3, b3)


def kernel(x, edge_index, batch,
           W_rel0, b_rel0, W_root0,
           W_rel1, b_rel1, W_root1,
           W_rel2, b_rel2, W_root2,
           lin1_W, lin1_b, lin2_W, lin2_b, lin3_W, lin3_b):
    src3 = edge_index[0].reshape(NW, NGRP, GRP, CPT)
    dst3 = edge_index[1].reshape(NW, NGRP, GRP, CPT)
    batch2d = batch.reshape(N, 1)

    acc = _seg_sum(x, src3, dst3)
    h1, s1, m1, cnt = _layer_call(acc, x, batch2d,
                                  W_rel0, b_rel0.reshape(1, D), W_root0)
    acc = _seg_sum(h1, src3, dst3)
    h2, s2, m2, _ = _layer_call(acc, h1, batch2d,
                                W_rel1, b_rel1.reshape(1, D), W_root1)
    acc = _seg_sum(h2, src3, dst3)
    h3, s3, m3, _ = _layer_call(acc, h2, batch2d,
                                W_rel2, b_rel2.reshape(1, D), W_root2)

    return _head_call(s1, s2, s3, m1, m2, m3, cnt,
                      lin1_W, lin1_b.reshape(1, G), lin2_W,
                      lin2_b.reshape(1, 32), lin3_W, lin3_b.reshape(1, OUT))


# speedup vs baseline: 9.7293x; 1.3426x over previous
import functools

import jax
import jax.numpy as jnp
from jax import lax
from jax.experimental import pallas as pl
from jax.experimental.pallas import tpu as pltpu
from jax.experimental.pallas import tpu_sc as plsc

N = 10000
E = 320000
D = 128
G = 64
OUT = 10

NC = 2
NS = 16
NW = NC * NS

CPT = 80
EPT = E // NW
NCH = EPT // CPT
GRP = 25
NGRP = NCH // GRP
NP = 10240
RPT = NP // NS

def _seg_sum_body(h_hbm, src_hbm, dst_hbm, out_hbm,
                  src_v, dst_v, rows0, rows1, acc_sh, sem0, sem1):
    cid = lax.axis_index("c")
    sid = lax.axis_index("s")
    wid = cid * NS + sid

    zz = jnp.zeros((16,), jnp.float32)

    def _zrow(r, carry):
        for j in range(D // 16):
            rows0[r, pl.ds(j * 16, 16)] = zz
        return carry

    lax.fori_loop(0, CPT, _zrow, 0)
    for t in range(RPT // CPT):
        pltpu.sync_copy(rows0,
                        acc_sh.at[pl.ds(sid * RPT + t * CPT, CPT)])
    plsc.subcore_barrier()

    for g in range(NGRP):
        pltpu.sync_copy(src_hbm.at[wid, g], src_v)
        pltpu.sync_copy(dst_hbm.at[wid, g], dst_v)

        pltpu.async_copy(h_hbm.at[src_v.at[0]], rows0, sem0)

        def _pair(p, carry):
            c0 = 2 * p
            c1 = c0 + 1
            pltpu.async_copy(h_hbm.at[src_v.at[c1]], rows1, sem1)
            pltpu.make_async_copy(h_hbm.at[src_v.at[c0]], rows0, sem0).wait()
            pltpu.sync_copy(rows0, acc_sh.at[dst_v.at[c0]], add=True)
            pltpu.async_copy(h_hbm.at[src_v.at[c1 + 1]], rows0, sem0)
            pltpu.make_async_copy(h_hbm.at[src_v.at[c1]], rows1, sem1).wait()
            pltpu.sync_copy(rows1, acc_sh.at[dst_v.at[c1]], add=True)
            return carry

        lax.fori_loop(0, (GRP - 1) // 2, _pair, 0)

        pltpu.make_async_copy(h_hbm.at[src_v.at[GRP - 1]], rows0, sem0).wait()
        pltpu.sync_copy(rows0, acc_sh.at[dst_v.at[GRP - 1]], add=True)

    plsc.subcore_barrier()
    pltpu.sync_copy(acc_sh.at[pl.ds(sid * RPT, RPT)],
                    out_hbm.at[cid, pl.ds(sid * RPT, RPT)])


@functools.cache
def _get_seg_sum():
    mesh = plsc.VectorSubcoreMesh(
        core_axis_name="c", subcore_axis_name="s",
        num_cores=NC, num_subcores=NS)
    return pl.kernel(
        _seg_sum_body,
        out_type=jax.ShapeDtypeStruct((NC, NP, D), jnp.float32),
        mesh=mesh,
        scratch_types=[
            pltpu.VMEM((GRP, CPT), jnp.int32),
            pltpu.VMEM((GRP, CPT), jnp.int32),
            pltpu.VMEM((CPT, D), jnp.float32),
            pltpu.VMEM((CPT, D), jnp.float32),
            pltpu.VMEM_SHARED((NP, D), jnp.float32),
            pltpu.SemaphoreType.DMA,
            pltpu.SemaphoreType.DMA,
        ],
    )


def _seg_sum(h, src3, dst3):
    return _get_seg_sum()(h, src3, dst3)


BN = 1000
NB = N // BN


def _layer_body(acc_ref, h_ref, batch_ref, wr_ref, br_ref, wo_ref,
                hn_ref, ps_ref, pm_ref, cnt_ref):
    i = pl.program_id(0)
    agg = acc_ref[0] + acc_ref[1]
    hn = jnp.dot(agg, wr_ref[...], preferred_element_type=jnp.float32)
    hn = hn + jnp.dot(h_ref[...], wo_ref[...], preferred_element_type=jnp.float32)
    hn = jnp.maximum(hn + br_ref[...], 0.0)
    hn_ref[...] = hn

    bb = batch_ref[...][:, 0]

    @pl.when(i == 0)
    def _():
        ps_ref[...] = jnp.zeros((G, D), jnp.float32)
        pm_ref[...] = jnp.zeros((G, D), jnp.float32)
        cnt_ref[...] = jnp.zeros((G, D), jnp.float32)

    lo = bb[0]
    hi = bb[BN - 1]
    gcol = lax.broadcasted_iota(jnp.int32, (G, 1), 0)

    def _g(g, carry):
        cur_s, cur_m, cur_c = carry
        mrow = (bb == g)[:, None]
        hm = jnp.where(mrow, hn, 0.0)
        s_g = jnp.sum(hm, axis=0)
        m_g = jnp.max(hm, axis=0)
        c_g = jnp.sum(mrow.astype(jnp.float32))
        sel = gcol == g
        cur_s = cur_s + jnp.where(sel, s_g[None, :], 0.0)
        cur_m = jnp.maximum(cur_m, jnp.where(sel, m_g[None, :], 0.0))
        cur_c = cur_c + jnp.where(sel, c_g, 0.0)
        return cur_s, cur_m, cur_c

    s0, m0, c0 = lax.fori_loop(
        lo, hi + 1, _g, (ps_ref[...], pm_ref[...], cnt_ref[...]))
    ps_ref[...] = s0
    pm_ref[...] = m0
    cnt_ref[...] = c0


def _layer_call(acc, h, batch2d, wr, br, wo):
    return pl.pallas_call(
        _layer_body,
        grid=(NB,),
        in_specs=[
            pl.BlockSpec((NC, BN, D), lambda i: (0, i, 0)),
            pl.BlockSpec((BN, D), lambda i: (i, 0)),
            pl.BlockSpec((BN, 1), lambda i: (i, 0)),
            pl.BlockSpec((D, D), lambda i: (0, 0)),
            pl.BlockSpec((1, D), lambda i: (0, 0)),
            pl.BlockSpec((D, D), lambda i: (0, 0)),
        ],
        out_specs=[
            pl.BlockSpec((BN, D), lambda i: (i, 0)),
            pl.BlockSpec((G, D), lambda i: (0, 0)),
            pl.BlockSpec((G, D), lambda i: (0, 0)),
            pl.BlockSpec((G, D), lambda i: (0, 0)),
        ],
        out_shape=[
            jax.ShapeDtypeStruct((N, D), jnp.float32),
            jax.ShapeDtypeStruct((G, D), jnp.float32),
            jax.ShapeDtypeStruct((G, D), jnp.float32),
            jax.ShapeDtypeStruct((G, D), jnp.float32),
        ],
    )(acc, h, batch2d, wr, br, wo)


def _head_body(s1, s2, s3, m1, m2, m3, cnt,
               w1, b1, w2, b2, w3, b3, out_ref):
    S = s1[...] + s2[...] + s3[...]
    M = m1[...] + m2[...] + m3[...]
    mean = S / jnp.maximum(cnt[...], 1.0)
    z = jnp.concatenate([mean, M], axis=1)
    z = jnp.maximum(
        jnp.dot(z, w1[...], preferred_element_type=jnp.float32) + b1[...], 0.0)
    z = jnp.maximum(
        jnp.dot(z, w2[...], preferred_element_type=jnp.float32) + b2[...], 0.0)
    out_ref[...] = (
        jnp.dot(z, w3[...], preferred_element_type=jnp.float32) + b3[...])


def _head_call(s1, s2, s3, m1, m2, m3, cnt, w1, b1, w2, b2, w3, b3):
    return pl.pallas_call(
        _head_body,
        out_shape=jax.ShapeDtypeStruct((G, OUT), jnp.float32),
    )(s1, s2, s3, m1, m2, m3, cnt, w1, b1, w2, b2, w3, b3)


def kernel(x, edge_index, batch,
           W_rel0, b_rel0, W_root0,
           W_rel1, b_rel1, W_root1,
           W_rel2, b_rel2, W_root2,
           lin1_W, lin1_b, lin2_W, lin2_b, lin3_W, lin3_b):
    src3 = edge_index[0].reshape(NW, NGRP, GRP, CPT)
    dst3 = edge_index[1].reshape(NW, NGRP, GRP, CPT)
    batch2d = batch.reshape(N, 1)

    acc = _seg_sum(x, src3, dst3)
    h1, s1, m1, cnt = _layer_call(acc, x, batch2d,
                                  W_rel0, b_rel0.reshape(1, D), W_root0)
    acc = _seg_sum(h1, src3, dst3)
    h2, s2, m2, _ = _layer_call(acc, h1, batch2d,
                                W_rel1, b_rel1.reshape(1, D), W_root1)
    acc = _seg_sum(h2, src3, dst3)
    h3, s3, m3, _ = _layer_call(acc, h2, batch2d,
                                W_rel2, b_rel2.reshape(1, D), W_root2)

    return _head_call(s1, s2, s3, m1, m2, m3, cnt,
                      lin1_W, lin1_b.reshape(1, G), lin2_W,
                      lin2_b.reshape(1, 32), lin3_W, lin3_b.reshape(1, OUT))
